# Initial kernel scaffold; baseline (speedup 1.0000x reference)
import jax
import jax.numpy as jnp
from jax.experimental import pallas as pl


def kernel(segment_center, outputs, label_inds):
    raise NotImplementedError("write your pallas kernel here")


# speedup vs baseline: 180.7015x; 180.7015x over previous
import functools

import jax
import jax.numpy as jnp
from jax import lax
from jax.experimental import pallas as pl
from jax.experimental.pallas import tpu as pltpu
from jax.experimental.pallas import tpu_sc as plsc

N_SEG, P, D, K, N_LABELS = 64, 512, 64, 15, 16
LANES = 16
NCHUNK = P // LANES


def _sc_body(center_hbm, outputs_hbm, labels_hbm, out_hbm,
             pts_v, cen_v, lab_v, dd_v, cnt_v, res_v):
    nc = 2
    wid = lax.axis_index("s") * nc + lax.axis_index("c")
    iota = lax.iota(jnp.int32, LANES)

    sum_dd_total = jnp.float32(0.0)
    push_total = jnp.int32(0)

    for s in range(2):
        seg = wid * 2 + s
        pltpu.sync_copy(outputs_hbm.at[seg], pts_v)
        pltpu.sync_copy(center_hbm.at[seg], cen_v)
        pltpu.sync_copy(labels_hbm.at[seg], lab_v)

        c_regs = [cen_v[pl.ds(16 * q, 16)] for q in range(4)]

        def dd_group(g, sacc):
            base = (g * LANES + iota) * D
            acc = jnp.zeros((LANES,), jnp.float32)
            for d in range(D):
                v = plsc.load_gather(pts_v, [base + d])
                t = v - c_regs[d // 16][d % 16]
                acc = acc + t * t
            dd_v[pl.ds(g * LANES, LANES)] = acc
            return sacc + acc

        sacc = lax.fori_loop(0, NCHUNK, dd_group,
                             jnp.zeros((LANES,), jnp.float32))
        sum_dd_total = sum_dd_total + jnp.sum(sacc)

        def chunk_body(g, carry):
            cnt_tab, macc = carry
            lv = lab_v[pl.ds(g * LANES, LANES)]
            ddc = dd_v[pl.ds(g * LANES, LANES)]
            for l in range(N_LABELS):
                on = lv == l
                pf = plsc.cumsum(on.astype(jnp.int32))
                base = cnt_tab[l]
                take = jnp.logical_and(on, (base + pf) <= K)
                macc = jnp.where(take, jnp.maximum(macc, ddc), macc)
                cnt_tab = cnt_tab + jnp.where(iota == l, pf[15], 0)
            return cnt_tab, macc

        cnt_tab, macc = lax.fori_loop(
            0, NCHUNK, chunk_body,
            (jnp.zeros((LANES,), jnp.int32),
             jnp.full((LANES,), -jnp.inf, jnp.float32)))

        lv0 = lab_v[pl.ds(0, LANES)]
        dd0 = dd_v[pl.ds(0, LANES)]
        for l in range(N_LABELS):
            c_l = cnt_tab[l]
            need = K - c_l
            active = jnp.logical_and(c_l > 0, need > 0)
            notl = lv0 != l
            pfn = plsc.cumsum(notl.astype(jnp.int32))
            take = jnp.logical_and(active,
                                   jnp.logical_and(notl, pfn <= need))
            macc = jnp.where(take, jnp.maximum(macc, dd0), macc)

        margin = jnp.float32(1.0) + jnp.max(macc)
        cnt_v[...] = cnt_tab

        def push_group(g, pacc):
            ddc = dd_v[pl.ds(g * LANES, LANES)]
            lv = lab_v[pl.ds(g * LANES, LANES)]
            cv = plsc.load_gather(cnt_v, [lv])
            w = jnp.where(ddc < margin, jnp.int32(P) - cv,
                          jnp.zeros((LANES,), jnp.int32))
            return pacc + w

        pacc = lax.fori_loop(0, NCHUNK, push_group,
                             jnp.zeros((LANES,), jnp.int32))
        push_total = push_total + jnp.sum(pacc)

    res = jnp.where(iota == 0, sum_dd_total,
                    jnp.where(iota == 1, push_total.astype(jnp.float32),
                              jnp.float32(0.0)))
    res_v[...] = res
    pltpu.sync_copy(res_v, out_hbm.at[wid])


@jax.jit
def _lmnn_sc(segment_center, outputs, label_inds):
    mesh = plsc.VectorSubcoreMesh(core_axis_name="c", subcore_axis_name="s")
    f = functools.partial(
        pl.kernel,
        out_type=jax.ShapeDtypeStruct((32, LANES), jnp.float32),
        mesh=mesh,
        compiler_params=pltpu.CompilerParams(needs_layout_passes=False),
        scratch_types=[
            pltpu.VMEM((P * D,), jnp.float32),
            pltpu.VMEM((D,), jnp.float32),
            pltpu.VMEM((P,), jnp.int32),
            pltpu.VMEM((P,), jnp.float32),
            pltpu.VMEM((N_LABELS,), jnp.int32),
            pltpu.VMEM((LANES,), jnp.float32),
        ],
    )(_sc_body)
    out = f(segment_center, outputs.reshape(N_SEG, P * D), label_inds)
    pull = jnp.float32(K) * jnp.sum(out[:, 0])
    push = jnp.sum(out[:, 1])
    return (pull + push) / jnp.float32(N_SEG * P)


def kernel(segment_center, outputs, label_inds):
    return _lmnn_sc(segment_center, outputs, label_inds)


# speedup vs baseline: 269.7834x; 1.4930x over previous
import functools

import jax
import jax.numpy as jnp
from jax import lax
from jax.experimental import pallas as pl
from jax.experimental.pallas import tpu as pltpu
from jax.experimental.pallas import tpu_sc as plsc

N_SEG, P, D, K, N_LABELS = 64, 512, 64, 15, 16
LANES = 16
NCHUNK = P // LANES


def _sc_body(center_hbm, outputs_hbm, labels_hbm, out_hbm,
             pts_v, cen_v, lab_v, dd_v, cnt_v, res_v):
    nc = 2
    wid = lax.axis_index("s") * nc + lax.axis_index("c")
    iota = lax.iota(jnp.int32, LANES)

    sum_dd_total = jnp.float32(0.0)
    push_total = jnp.int32(0)

    for s in range(2):
        seg = wid * 2 + s
        pltpu.sync_copy(outputs_hbm.at[seg], pts_v)
        pltpu.sync_copy(center_hbm.at[seg], cen_v)
        pltpu.sync_copy(labels_hbm.at[seg], lab_v)

        c_regs = [cen_v[pl.ds(16 * q, 16)] for q in range(4)]

        def dd_group(g, sacc):
            base = g * LANES
            acc = jnp.zeros((LANES,), jnp.float32)
            for j in range(LANES):
                row = (base + j) * D
                t0 = pts_v[pl.ds(row, 16)] - c_regs[0]
                t1 = pts_v[pl.ds(row + 16, 16)] - c_regs[1]
                t2 = pts_v[pl.ds(row + 32, 16)] - c_regs[2]
                t3 = pts_v[pl.ds(row + 48, 16)] - c_regs[3]
                s = t0 * t0 + t1 * t1 + t2 * t2 + t3 * t3
                acc = jnp.where(iota == j, jnp.sum(s), acc)
            dd_v[pl.ds(base, LANES)] = acc
            return sacc + acc

        sacc = lax.fori_loop(0, NCHUNK, dd_group,
                             jnp.zeros((LANES,), jnp.float32))
        sum_dd_total = sum_dd_total + jnp.sum(sacc)

        def chunk_body(g, carry):
            cnt_tab, macc = carry
            lv = lab_v[pl.ds(g * LANES, LANES)]
            ddc = dd_v[pl.ds(g * LANES, LANES)]
            for l in range(N_LABELS):
                on = lv == l
                pf = plsc.cumsum(on.astype(jnp.int32))
                base = cnt_tab[l]
                take = jnp.logical_and(on, (base + pf) <= K)
                macc = jnp.where(take, jnp.maximum(macc, ddc), macc)
                cnt_tab = cnt_tab + jnp.where(iota == l, pf[15], 0)
            return cnt_tab, macc

        cnt_tab, macc = lax.fori_loop(
            0, NCHUNK, chunk_body,
            (jnp.zeros((LANES,), jnp.int32),
             jnp.full((LANES,), -jnp.inf, jnp.float32)))

        lv0 = lab_v[pl.ds(0, LANES)]
        dd0 = dd_v[pl.ds(0, LANES)]
        for l in range(N_LABELS):
            c_l = cnt_tab[l]
            need = K - c_l
            active = jnp.logical_and(c_l > 0, need > 0)
            notl = lv0 != l
            pfn = plsc.cumsum(notl.astype(jnp.int32))
            take = jnp.logical_and(active,
                                   jnp.logical_and(notl, pfn <= need))
            macc = jnp.where(take, jnp.maximum(macc, dd0), macc)

        margin = jnp.float32(1.0) + jnp.max(macc)
        cnt_v[...] = cnt_tab

        def push_group(g, pacc):
            ddc = dd_v[pl.ds(g * LANES, LANES)]
            lv = lab_v[pl.ds(g * LANES, LANES)]
            cv = plsc.load_gather(cnt_v, [lv])
            w = jnp.where(ddc < margin, jnp.int32(P) - cv,
                          jnp.zeros((LANES,), jnp.int32))
            return pacc + w

        pacc = lax.fori_loop(0, NCHUNK, push_group,
                             jnp.zeros((LANES,), jnp.int32))
        push_total = push_total + jnp.sum(pacc)

    res = jnp.where(iota == 0, sum_dd_total,
                    jnp.where(iota == 1, push_total.astype(jnp.float32),
                              jnp.float32(0.0)))
    res_v[...] = res
    pltpu.sync_copy(res_v, out_hbm.at[wid])


@jax.jit
def _lmnn_sc(segment_center, outputs, label_inds):
    mesh = plsc.VectorSubcoreMesh(core_axis_name="c", subcore_axis_name="s")
    f = functools.partial(
        pl.kernel,
        out_type=jax.ShapeDtypeStruct((32, LANES), jnp.float32),
        mesh=mesh,
        compiler_params=pltpu.CompilerParams(needs_layout_passes=False),
        scratch_types=[
            pltpu.VMEM((P * D,), jnp.float32),
            pltpu.VMEM((D,), jnp.float32),
            pltpu.VMEM((P,), jnp.int32),
            pltpu.VMEM((P,), jnp.float32),
            pltpu.VMEM((N_LABELS,), jnp.int32),
            pltpu.VMEM((LANES,), jnp.float32),
        ],
    )(_sc_body)
    out = f(segment_center, outputs.reshape(N_SEG, P * D), label_inds)
    pull = jnp.float32(K) * jnp.sum(out[:, 0])
    push = jnp.sum(out[:, 1])
    return (pull + push) / jnp.float32(N_SEG * P)


def kernel(segment_center, outputs, label_inds):
    return _lmnn_sc(segment_center, outputs, label_inds)


# speedup vs baseline: 319.7725x; 1.1853x over previous
import functools

import jax
import jax.numpy as jnp
from jax import lax
from jax.experimental import pallas as pl
from jax.experimental.pallas import tpu as pltpu
from jax.experimental.pallas import tpu_sc as plsc

N_SEG, P, D, K, N_LABELS = 64, 512, 64, 15, 16
LANES = 16
NCHUNK = P // LANES


def _sc_body(center_hbm, outputs_hbm, labels_hbm, out_hbm,
             pts_v, cen_v, lab_v, dd_v, cnt_v, res_v):
    nc = 2
    wid = lax.axis_index("s") * nc + lax.axis_index("c")
    iota = lax.iota(jnp.int32, LANES)

    sum_dd_total = jnp.float32(0.0)
    push_total = jnp.int32(0)

    for s in range(2):
        seg = wid * 2 + s
        pltpu.sync_copy(outputs_hbm.at[seg], pts_v)
        pltpu.sync_copy(center_hbm.at[seg], cen_v)
        pltpu.sync_copy(labels_hbm.at[seg], lab_v)

        c_regs = [cen_v[pl.ds(16 * q, 16)] for q in range(4)]

        def dd_group(g, sacc):
            base = g * LANES
            acc = jnp.zeros((LANES,), jnp.float32)
            for j in range(LANES):
                row = base + j
                t0 = pts_v[row, pl.ds(0, 16)] - c_regs[0]
                t1 = pts_v[row, pl.ds(16, 16)] - c_regs[1]
                t2 = pts_v[row, pl.ds(32, 16)] - c_regs[2]
                t3 = pts_v[row, pl.ds(48, 16)] - c_regs[3]
                s = t0 * t0 + t1 * t1 + t2 * t2 + t3 * t3
                acc = jnp.where(iota == j, jnp.sum(s), acc)
            dd_v[pl.ds(base, LANES)] = acc
            return sacc + acc

        sacc = lax.fori_loop(0, NCHUNK, dd_group,
                             jnp.zeros((LANES,), jnp.float32))
        sum_dd_total = sum_dd_total + jnp.sum(sacc)

        def chunk_body(g, carry):
            cnt_tab, macc = carry
            lv = lab_v[pl.ds(g * LANES, LANES)]
            ddc = dd_v[pl.ds(g * LANES, LANES)]
            for l in range(N_LABELS):
                on = lv == l
                pf = plsc.cumsum(on.astype(jnp.int32))
                base = cnt_tab[l]
                take = jnp.logical_and(on, (base + pf) <= K)
                macc = jnp.where(take, jnp.maximum(macc, ddc), macc)
                cnt_tab = cnt_tab + jnp.where(iota == l, pf[15], 0)
            return cnt_tab, macc

        cnt_tab, macc = lax.fori_loop(
            0, NCHUNK, chunk_body,
            (jnp.zeros((LANES,), jnp.int32),
             jnp.full((LANES,), -jnp.inf, jnp.float32)))

        lv0 = lab_v[pl.ds(0, LANES)]
        dd0 = dd_v[pl.ds(0, LANES)]
        for l in range(N_LABELS):
            c_l = cnt_tab[l]
            need = K - c_l
            active = jnp.logical_and(c_l > 0, need > 0)
            notl = lv0 != l
            pfn = plsc.cumsum(notl.astype(jnp.int32))
            take = jnp.logical_and(active,
                                   jnp.logical_and(notl, pfn <= need))
            macc = jnp.where(take, jnp.maximum(macc, dd0), macc)

        margin = jnp.float32(1.0) + jnp.max(macc)
        cnt_v[...] = cnt_tab

        def push_group(g, pacc):
            ddc = dd_v[pl.ds(g * LANES, LANES)]
            lv = lab_v[pl.ds(g * LANES, LANES)]
            cv = plsc.load_gather(cnt_v, [lv])
            w = jnp.where(ddc < margin, jnp.int32(P) - cv,
                          jnp.zeros((LANES,), jnp.int32))
            return pacc + w

        pacc = lax.fori_loop(0, NCHUNK, push_group,
                             jnp.zeros((LANES,), jnp.int32))
        push_total = push_total + jnp.sum(pacc)

    res = jnp.where(iota == 0, sum_dd_total,
                    jnp.where(iota == 1, push_total.astype(jnp.float32),
                              jnp.float32(0.0)))
    res_v[...] = res
    pltpu.sync_copy(res_v, out_hbm.at[wid])


@jax.jit
def _lmnn_sc(segment_center, outputs, label_inds):
    mesh = plsc.VectorSubcoreMesh(core_axis_name="c", subcore_axis_name="s")
    f = functools.partial(
        pl.kernel,
        out_type=jax.ShapeDtypeStruct((32, LANES), jnp.float32),
        mesh=mesh,
        compiler_params=pltpu.CompilerParams(needs_layout_passes=False),
        scratch_types=[
            pltpu.VMEM((P, D), jnp.float32),
            pltpu.VMEM((D,), jnp.float32),
            pltpu.VMEM((P,), jnp.int32),
            pltpu.VMEM((P,), jnp.float32),
            pltpu.VMEM((N_LABELS,), jnp.int32),
            pltpu.VMEM((LANES,), jnp.float32),
        ],
    )(_sc_body)
    out = f(segment_center, outputs, label_inds)
    pull = jnp.float32(K) * jnp.sum(out[:, 0])
    push = jnp.sum(out[:, 1])
    return (pull + push) / jnp.float32(N_SEG * P)


def kernel(segment_center, outputs, label_inds):
    return _lmnn_sc(segment_center, outputs, label_inds)
